# baseline (speedup 1.0000x reference)
import jax
import jax.numpy as jnp
from jax import lax
from jax.experimental import pallas as pl
from jax.experimental.pallas import tpu as pltpu
from jax.experimental.pallas import tpu_sc as plsc

N_TOKENS = 8192
FEAT = 4096
LANES = 16

NUM_CORES = 2
NUM_SUBCORES = 16
NUM_WORKERS = NUM_CORES * NUM_SUBCORES
ROWS_PER_WORKER = N_TOKENS // NUM_WORKERS

IN_ROWS = 8
OUT_ROWS = 4
HALVES = IN_ROWS // OUT_ROWS
NUM_IN_BLOCKS = ROWS_PER_WORKER // IN_ROWS
NBUF_IN = 2


def _permute_body(x_hbm, perm_hbm, out_hbm, perm_v,
                  in0, in1, out0, out1,
                  sin0, sin1, sout0, sout1):
    wid = lax.axis_index("s") * NUM_CORES + lax.axis_index("c")
    base = wid * ROWS_PER_WORKER

    pltpu.sync_copy(perm_hbm, perm_v)

    ins, sins = [in0, in1], [sin0, sin1]
    outs, souts = [out0, out1], [sout0, sout1]

    def x_slice(ib):
        return x_hbm.at[pl.ds(base + ib * IN_ROWS, IN_ROWS)]

    def y_slice(ib, h):
        return out_hbm.at[pl.ds(base + ib * IN_ROWS + h * OUT_ROWS, OUT_ROWS)]

    row_ids = [jnp.full((LANES,), r, jnp.int32) for r in range(IN_ROWS)]

    def gather_half(in_v, h, out_v):
        @plsc.parallel_loop(0, FEAT // LANES, 1, unroll=8)
        def jbody(j):
            idx = perm_v[pl.ds(j * LANES, LANES)]
            for r in range(OUT_ROWS):
                out_v[r, pl.ds(j * LANES, LANES)] = plsc.load_gather(
                    in_v, [row_ids[h * OUT_ROWS + r], idx])

    for bi in range(NBUF_IN):
        pltpu.async_copy(x_slice(bi), ins[bi], sins[bi])

    def outer(t, carry):
        for bi in range(NBUF_IN):
            ib = t * NBUF_IN + bi
            pltpu.make_async_copy(x_slice(ib), ins[bi], sins[bi]).wait()

            for h in range(HALVES):
                @pl.when(ib > 0)
                def _wait_out():
                    pltpu.make_async_copy(
                        outs[h], y_slice(ib - 1, h), souts[h]).wait()

                gather_half(ins[bi], h, outs[h])
                pltpu.async_copy(outs[h], y_slice(ib, h), souts[h])

            @pl.when(ib + NBUF_IN < NUM_IN_BLOCKS)
            def _next_in():
                pltpu.async_copy(x_slice(ib + NBUF_IN), ins[bi], sins[bi])
        return carry

    lax.fori_loop(0, NUM_IN_BLOCKS // NBUF_IN, outer, 0)

    for h in range(HALVES):
        pltpu.make_async_copy(
            outs[h], y_slice(NUM_IN_BLOCKS - 1, h), souts[h]).wait()


@jax.jit
def kernel(x, permute):
    perm = permute.astype(jnp.int32)
    mesh = plsc.VectorSubcoreMesh(
        core_axis_name="c", subcore_axis_name="s",
        num_cores=NUM_CORES, num_subcores=NUM_SUBCORES)
    run = pl.kernel(
        _permute_body,
        out_type=jax.ShapeDtypeStruct((N_TOKENS, FEAT), jnp.float32),
        mesh=mesh,
        compiler_params=pltpu.CompilerParams(needs_layout_passes=False),
        scratch_types=[
            pltpu.VMEM((FEAT,), jnp.int32),
            pltpu.VMEM((IN_ROWS, FEAT), jnp.float32),
            pltpu.VMEM((IN_ROWS, FEAT), jnp.float32),
            pltpu.VMEM((OUT_ROWS, FEAT), jnp.float32),
            pltpu.VMEM((OUT_ROWS, FEAT), jnp.float32),
            pltpu.SemaphoreType.DMA,
            pltpu.SemaphoreType.DMA,
            pltpu.SemaphoreType.DMA,
            pltpu.SemaphoreType.DMA,
        ],
    )
    return run(x, perm)


# speedup vs baseline: 1.0038x; 1.0038x over previous
import jax
import jax.numpy as jnp
from jax import lax
from jax.experimental import pallas as pl
from jax.experimental.pallas import tpu as pltpu
from jax.experimental.pallas import tpu_sc as plsc

N_TOKENS = 8192
FEAT = 4096
LANES = 16

NUM_CORES = 2
NUM_SUBCORES = 16
NUM_WORKERS = NUM_CORES * NUM_SUBCORES
ROWS_PER_WORKER = N_TOKENS // NUM_WORKERS
ROWS_PER_BLOCK = 4
NUM_BLOCKS = ROWS_PER_WORKER // ROWS_PER_BLOCK
NBUF_IN = 4
NBUF_OUT = 2


def _permute_body(x_hbm, perm_hbm, out_hbm, perm_v,
                  in0, in1, in2, in3, out0, out1,
                  sin0, sin1, sin2, sin3, sout0, sout1):
    wid = lax.axis_index("s") * NUM_CORES + lax.axis_index("c")
    base = wid * ROWS_PER_WORKER

    pltpu.sync_copy(perm_hbm, perm_v)

    ins, sins = [in0, in1, in2, in3], [sin0, sin1, sin2, sin3]
    outs, souts = [out0, out1], [sout0, sout1]

    def x_slice(g):
        return x_hbm.at[pl.ds(base + g * ROWS_PER_BLOCK, ROWS_PER_BLOCK)]

    def y_slice(g):
        return out_hbm.at[pl.ds(base + g * ROWS_PER_BLOCK, ROWS_PER_BLOCK)]

    row_ids = [jnp.full((LANES,), r, jnp.int32) for r in range(ROWS_PER_BLOCK)]

    def gather_block(in_v, out_v):
        @plsc.parallel_loop(0, FEAT // LANES, 1, unroll=8)
        def jbody(j):
            idx = perm_v[pl.ds(j * LANES, LANES)]
            for r in range(ROWS_PER_BLOCK):
                out_v[r, pl.ds(j * LANES, LANES)] = plsc.load_gather(
                    in_v, [row_ids[r], idx])

    for bi in range(NBUF_IN):
        pltpu.async_copy(x_slice(bi), ins[bi], sins[bi])

    def outer(t, carry):
        for bi in range(NBUF_IN):
            g = t * NBUF_IN + bi
            bo = bi % NBUF_OUT
            pltpu.make_async_copy(x_slice(g), ins[bi], sins[bi]).wait()

            @pl.when(g >= NBUF_OUT)
            def _wait_out():
                pltpu.make_async_copy(
                    outs[bo], y_slice(g - NBUF_OUT), souts[bo]).wait()

            gather_block(ins[bi], outs[bo])
            pltpu.async_copy(outs[bo], y_slice(g), souts[bo])

            @pl.when(g + NBUF_IN < NUM_BLOCKS)
            def _next_in():
                pltpu.async_copy(x_slice(g + NBUF_IN), ins[bi], sins[bi])
        return carry

    lax.fori_loop(0, NUM_BLOCKS // NBUF_IN, outer, 0)

    for bo in range(NBUF_OUT):
        g = NUM_BLOCKS - NBUF_OUT + bo
        pltpu.make_async_copy(outs[bo], y_slice(g), souts[bo]).wait()


@jax.jit
def kernel(x, permute):
    perm = permute.astype(jnp.int32)
    mesh = plsc.VectorSubcoreMesh(
        core_axis_name="c", subcore_axis_name="s",
        num_cores=NUM_CORES, num_subcores=NUM_SUBCORES)
    run = pl.kernel(
        _permute_body,
        out_type=jax.ShapeDtypeStruct((N_TOKENS, FEAT), jnp.float32),
        mesh=mesh,
        compiler_params=pltpu.CompilerParams(needs_layout_passes=False),
        scratch_types=[
            pltpu.VMEM((FEAT,), jnp.int32),
            pltpu.VMEM((ROWS_PER_BLOCK, FEAT), jnp.float32),
            pltpu.VMEM((ROWS_PER_BLOCK, FEAT), jnp.float32),
            pltpu.VMEM((ROWS_PER_BLOCK, FEAT), jnp.float32),
            pltpu.VMEM((ROWS_PER_BLOCK, FEAT), jnp.float32),
            pltpu.VMEM((ROWS_PER_BLOCK, FEAT), jnp.float32),
            pltpu.VMEM((ROWS_PER_BLOCK, FEAT), jnp.float32),
            pltpu.SemaphoreType.DMA,
            pltpu.SemaphoreType.DMA,
            pltpu.SemaphoreType.DMA,
            pltpu.SemaphoreType.DMA,
            pltpu.SemaphoreType.DMA,
            pltpu.SemaphoreType.DMA,
        ],
    )
    return run(x, perm)


# speedup vs baseline: 1.0075x; 1.0037x over previous
import jax
import jax.numpy as jnp
from jax import lax
from jax.experimental import pallas as pl
from jax.experimental.pallas import tpu as pltpu
from jax.experimental.pallas import tpu_sc as plsc

N_TOKENS = 8192
FEAT = 4096
LANES = 16

NUM_CORES = 2
NUM_SUBCORES = 16
NUM_WORKERS = NUM_CORES * NUM_SUBCORES
ROWS_PER_WORKER = N_TOKENS // NUM_WORKERS
ROWS_PER_BLOCK = 4
NUM_BLOCKS = ROWS_PER_WORKER // ROWS_PER_BLOCK
NBUF_IN = 4
NBUF_OUT = 2


def _permute_body(x_hbm, perm_hbm, out_hbm, perm_v,
                  in0, in1, in2, in3, out0, out1,
                  sin0, sin1, sin2, sin3, sout0, sout1):
    wid = lax.axis_index("s") * NUM_CORES + lax.axis_index("c")
    base = wid * ROWS_PER_WORKER

    pltpu.sync_copy(perm_hbm, perm_v)

    ins, sins = [in0, in1, in2, in3], [sin0, sin1, sin2, sin3]
    outs, souts = [out0, out1], [sout0, sout1]

    def x_slice(g):
        return x_hbm.at[pl.ds(base + g * ROWS_PER_BLOCK, ROWS_PER_BLOCK)]

    def y_slice(g):
        return out_hbm.at[pl.ds(base + g * ROWS_PER_BLOCK, ROWS_PER_BLOCK)]

    row_ids = [jnp.full((LANES,), r, jnp.int32) for r in range(ROWS_PER_BLOCK)]

    def gather_block(in_v, out_v):
        @plsc.parallel_loop(0, FEAT // LANES, 1, unroll=8)
        def jbody(j):
            idx = perm_v[pl.ds(j * LANES, LANES)]
            for r in range(ROWS_PER_BLOCK):
                out_v[r, pl.ds(j * LANES, LANES)] = plsc.load_gather(
                    in_v, [row_ids[r], idx])

    for bi in range(NBUF_IN):
        pltpu.async_copy(x_slice(bi), ins[bi], sins[bi])

    def outer(t, carry):
        for bi in range(NBUF_IN):
            g = t * NBUF_IN + bi
            bo = bi % NBUF_OUT
            pltpu.make_async_copy(x_slice(g), ins[bi], sins[bi]).wait()

            @pl.when(g >= NBUF_OUT)
            def _wait_out():
                pltpu.make_async_copy(
                    outs[bo], y_slice(g - NBUF_OUT), souts[bo]).wait()

            gather_block(ins[bi], outs[bo])
            pltpu.async_copy(outs[bo], y_slice(g), souts[bo])

            @pl.when(g + NBUF_IN < NUM_BLOCKS)
            def _next_in():
                pltpu.async_copy(x_slice(g + NBUF_IN), ins[bi], sins[bi])
        return carry

    lax.fori_loop(0, NUM_BLOCKS // NBUF_IN, outer, 0)

    for bo in range(NBUF_OUT):
        g = NUM_BLOCKS - NBUF_OUT + bo
        pltpu.make_async_copy(outs[bo], y_slice(g), souts[bo]).wait()


@jax.jit
def kernel(x, permute):
    perm = permute.astype(jnp.int32)
    mesh = plsc.VectorSubcoreMesh(
        core_axis_name="c", subcore_axis_name="s",
        num_cores=NUM_CORES, num_subcores=NUM_SUBCORES)
    run = pl.kernel(
        _permute_body,
        out_type=jax.ShapeDtypeStruct((N_TOKENS, FEAT), jnp.float32),
        mesh=mesh,
        compiler_params=pltpu.CompilerParams(needs_layout_passes=False, skip_device_barrier=True),
        scratch_types=[
            pltpu.VMEM((FEAT,), jnp.int32),
            pltpu.VMEM((ROWS_PER_BLOCK, FEAT), jnp.float32),
            pltpu.VMEM((ROWS_PER_BLOCK, FEAT), jnp.float32),
            pltpu.VMEM((ROWS_PER_BLOCK, FEAT), jnp.float32),
            pltpu.VMEM((ROWS_PER_BLOCK, FEAT), jnp.float32),
            pltpu.VMEM((ROWS_PER_BLOCK, FEAT), jnp.float32),
            pltpu.VMEM((ROWS_PER_BLOCK, FEAT), jnp.float32),
            pltpu.SemaphoreType.DMA,
            pltpu.SemaphoreType.DMA,
            pltpu.SemaphoreType.DMA,
            pltpu.SemaphoreType.DMA,
            pltpu.SemaphoreType.DMA,
            pltpu.SemaphoreType.DMA,
        ],
    )
    return run(x, perm)
